# baseline (speedup 1.0000x reference)
import dataclasses
import functools

import jax
import jax.numpy as jnp
from jax import lax
from jax.experimental import pallas as pl
from jax.experimental.pallas import tpu as pltpu
from jax.experimental.pallas import tpu_sc as plsc

N = 10000
DIN = 128
E = 320000
NC = 2
NS = 16
L = 16
DH = DIN // NC
K = 80
NBUF = 5
R = N
RPT = R // NS
NCHUNK = 250
EPT = NCHUNK * K
HR = 80
NFULL = K // L

_CP = pltpu.CompilerParams(use_tc_tiling_on_sc=False)
if "needs_layout_passes" in pltpu.CompilerParams.__dataclass_fields__:
    _CP = dataclasses.replace(_CP, needs_layout_passes=False)


def _sc_aggregate(x, edges, zeros_tile, iota_hr):
    mesh = plsc.VectorSubcoreMesh(core_axis_name="c", subcore_axis_name="s")

    @functools.partial(
        pl.kernel,
        mesh=mesh,
        compiler_params=_CP,
        out_type=[jax.ShapeDtypeStruct((NC, R, DH), jnp.float32),
                  jax.ShapeDtypeStruct((NC, HR, 128), jnp.float32),
                  jax.ShapeDtypeStruct((NC, N, DH), jnp.float32)],
        scratch_types=[
            pltpu.VMEM((EPT,), jnp.int32),
            pltpu.VMEM((EPT,), jnp.int32),
            *[pltpu.VMEM((K, DH), jnp.float32) for _ in range(NBUF)],
            pltpu.VMEM((HR, 128), jnp.float32),
            pltpu.VMEM((HR,), jnp.int32),
            pltpu.VMEM_SHARED((R, DH), jnp.float32),
            pltpu.VMEM_SHARED((HR, 128), jnp.float32),
            *[pltpu.SemaphoreType.DMA for _ in range(2 * NBUF)],
        ],
    )
    def body(x_hbm, e_hbm, z_hbm, i_hbm, outa_hbm, outd_hbm, xh_hbm,
             sidx, didx, *bufs):
        rows = bufs[:NBUF]
        hist = bufs[NBUF]
        iota_v = bufs[NBUF + 1]
        agg = bufs[NBUF + 2]
        sdeg = bufs[NBUF + 3]
        gsem = bufs[NBUF + 4:NBUF + 4 + NBUF]
        ssem = bufs[NBUF + 4 + NBUF:]
        c = lax.axis_index("c")
        s = lax.axis_index("s")
        xh = xh_hbm.at[c]
        ones16 = jnp.ones((L,), jnp.float32)
        zeros16 = jnp.zeros((L,), jnp.float32)

        cp0 = pltpu.async_copy(e_hbm.at[0, pl.ds(s * EPT, EPT)], sidx, gsem[0])
        cp1 = pltpu.async_copy(e_hbm.at[1, pl.ds(s * EPT, EPT)], didx, gsem[1])
        cp2 = pltpu.async_copy(z_hbm, agg.at[pl.ds(s * RPT, RPT)], gsem[2])
        cp3 = pltpu.async_copy(i_hbm, iota_v, gsem[3])

        @pl.loop(0, HR)
        def _(r):
            for v in range(8):
                hist[r, pl.ds(v * L, L)] = zeros16

        cp0.wait()
        cp1.wait()
        cp2.wait()
        cp3.wait()
        base = s * RPT
        for q in range(RPT // K):
            sync0 = pltpu.sync_copy
            sync0(x_hbm.at[pl.ds(base + q * K, K), pl.ds(c * DH, DH)], rows[0])
            sync0(rows[0], xh.at[pl.ds(base + q * K, K)])
        tail = RPT - (RPT // K) * K
        if tail:
            toff = base + (RPT // K) * K
            pltpu.sync_copy(
                x_hbm.at[pl.ds(toff, tail), pl.ds(c * DH, DH)],
                rows[0].at[pl.ds(0, tail)])
            pltpu.sync_copy(rows[0].at[pl.ds(0, tail)],
                            xh.at[pl.ds(toff, tail)])

        @pl.when(s == 0)
        def _():
            pltpu.sync_copy(hist, sdeg)
        plsc.subcore_barrier()

        def hist_row(r):
            for v in range(NFULL):
                idx = didx[pl.ds(r * K + v * L, L)]
                plsc.addupdate_scatter(hist, [idx >> 7, idx & 127], ones16)

        for b in range(NBUF):
            pltpu.async_copy(xh.at[sidx.at[pl.ds(b * K, K)]], rows[b], gsem[b])

        @pl.loop(0, NCHUNK, step=NBUF)
        def _(j):
            for b in range(NBUF):
                pltpu.make_async_copy(xh.at[sidx.at[pl.ds(0, K)]], rows[b],
                                      gsem[b]).wait()
                pltpu.async_copy(rows[b], agg.at[didx.at[pl.ds((j + b) * K, K)]],
                                 ssem[b], add=True)
            for b in range(NBUF):
                hist_row(j + b)
            for b in range(NBUF):
                pltpu.make_async_copy(rows[b], agg.at[didx.at[pl.ds(0, K)]],
                                      ssem[b]).wait()

                @pl.when(j + NBUF + b < NCHUNK)
                def _():
                    pltpu.async_copy(
                        xh.at[sidx.at[pl.ds((j + NBUF + b) * K, K)]], rows[b],
                        gsem[b])

        plsc.subcore_barrier()
        pltpu.sync_copy(hist, sdeg.at[iota_v], add=True)
        plsc.subcore_barrier()
        pltpu.sync_copy(agg.at[pl.ds(s * RPT, RPT)],
                        outa_hbm.at[c, pl.ds(s * RPT, RPT)])
        pltpu.sync_copy(sdeg.at[pl.ds(s * (HR // NS), HR // NS)],
                        outd_hbm.at[c, pl.ds(s * (HR // NS), HR // NS)])

    return body(x, edges, zeros_tile, iota_hr)


def _tc_root(x, wrT, bias):
    BR = 1024
    nblk = (N + BR - 1) // BR

    def body(x_ref, wr_ref, b_ref, o_ref):
        o_ref[...] = jnp.dot(
            x_ref[...], wr_ref[...],
            preferred_element_type=jnp.float32) + b_ref[...]

    return pl.pallas_call(
        body,
        grid=(nblk,),
        in_specs=[
            pl.BlockSpec((BR, DIN), lambda i: (i, 0)),
            pl.BlockSpec((DIN, DIN), lambda i: (0, 0)),
            pl.BlockSpec((1, DIN), lambda i: (0, 0)),
        ],
        out_specs=pl.BlockSpec((BR, DIN), lambda i: (i, 0)),
        out_shape=jax.ShapeDtypeStruct((N, DIN), jnp.float32),
    )(x, wrT, bias)


def _tc_combine(agg, deg, root, wnT):
    BR = 2048
    nblk = (N + BR - 1) // BR
    DR = BR // 128

    def body(a_ref, d_ref, r_ref, wn_ref, o_ref):
        dcol = jnp.concatenate(
            [jnp.transpose(d_ref[0, i:i + 1, :]) for i in range(DR)], axis=0)
        d = jnp.maximum(dcol, 1.0)
        neigh = jnp.concatenate([a_ref[0], a_ref[1]], axis=1) / d
        o_ref[...] = jnp.dot(
            neigh.astype(jnp.bfloat16), wn_ref[...],
            preferred_element_type=jnp.float32) + r_ref[...]

    return pl.pallas_call(
        body,
        grid=(nblk,),
        in_specs=[
            pl.BlockSpec((NC, BR, DH), lambda i: (0, i, 0)),
            pl.BlockSpec((1, DR, 128), lambda i: (0, i, 0)),
            pl.BlockSpec((BR, DIN), lambda i: (i, 0)),
            pl.BlockSpec((DIN, DIN), lambda i: (0, 0)),
        ],
        out_specs=pl.BlockSpec((BR, DIN), lambda i: (i, 0)),
        out_shape=jax.ShapeDtypeStruct((N, DIN), jnp.float32),
    )(agg, deg, root, wnT.astype(jnp.bfloat16))


def kernel(x, edge_index, W_neigh, b_neigh, W_root, b_root):
    f32 = jnp.float32
    edges = edge_index.astype(jnp.int32)
    x = x.astype(f32)
    zeros_tile = jnp.zeros((RPT, DH), f32)
    iota_hr = jnp.arange(HR, dtype=jnp.int32)
    agg, deg, _ = _sc_aggregate(x, edges, zeros_tile, iota_hr)
    bias = (b_neigh + b_root).reshape(1, DIN).astype(f32)
    root = _tc_root(x, W_root.T.astype(f32), bias)
    return _tc_combine(agg, deg, root, W_neigh.T.astype(f32))


# speedup vs baseline: 1.0881x; 1.0881x over previous
import dataclasses
import functools

import jax
import jax.numpy as jnp
from jax import lax
from jax.experimental import pallas as pl
from jax.experimental.pallas import tpu as pltpu
from jax.experimental.pallas import tpu_sc as plsc

N = 10000
DIN = 128
E = 320000
NC = 2
NS = 16
L = 16
DH = DIN // NC
K = 80
NBUF = 5
R = 10240
RPT = R // NS
XPT = N // NS
NCHUNK = 250
EPT = NCHUNK * K
HR = 80
NFULL = K // L

_CP = pltpu.CompilerParams(use_tc_tiling_on_sc=False)
if "needs_layout_passes" in pltpu.CompilerParams.__dataclass_fields__:
    _CP = dataclasses.replace(_CP, needs_layout_passes=False)


def _sc_aggregate(x, edges, zeros_tile, iota_hr):
    mesh = plsc.VectorSubcoreMesh(core_axis_name="c", subcore_axis_name="s")

    @functools.partial(
        pl.kernel,
        mesh=mesh,
        compiler_params=_CP,
        out_type=[jax.ShapeDtypeStruct((R, DIN), jnp.float32),
                  jax.ShapeDtypeStruct((NC, HR, 128), jnp.float32),
                  jax.ShapeDtypeStruct((NC, N, DH), jnp.float32)],
        scratch_types=[
            pltpu.VMEM((EPT,), jnp.int32),
            pltpu.VMEM((EPT,), jnp.int32),
            *[pltpu.VMEM((K, DH), jnp.float32) for _ in range(NBUF)],
            pltpu.VMEM((HR, 128), jnp.float32),
            pltpu.VMEM((HR,), jnp.int32),
            pltpu.VMEM_SHARED((R, DH), jnp.float32),
            pltpu.VMEM_SHARED((HR, 128), jnp.float32),
            *[pltpu.SemaphoreType.DMA for _ in range(2 * NBUF)],
        ],
    )
    def body(x_hbm, e_hbm, z_hbm, i_hbm, outa_hbm, outd_hbm, xh_hbm,
             sidx, didx, *bufs):
        rows = bufs[:NBUF]
        hist = bufs[NBUF]
        iota_v = bufs[NBUF + 1]
        agg = bufs[NBUF + 2]
        sdeg = bufs[NBUF + 3]
        gsem = bufs[NBUF + 4:NBUF + 4 + NBUF]
        ssem = bufs[NBUF + 4 + NBUF:]
        c = lax.axis_index("c")
        s = lax.axis_index("s")
        xh = xh_hbm.at[c]
        ones16 = jnp.ones((L,), jnp.float32)
        zeros16 = jnp.zeros((L,), jnp.float32)

        cp0 = pltpu.async_copy(e_hbm.at[0, pl.ds(s * EPT, EPT)], sidx, gsem[0])
        cp1 = pltpu.async_copy(e_hbm.at[1, pl.ds(s * EPT, EPT)], didx, gsem[1])
        cp2 = pltpu.async_copy(z_hbm, agg.at[pl.ds(s * RPT, RPT)], gsem[2])
        cp3 = pltpu.async_copy(i_hbm, iota_v, gsem[3])

        @pl.loop(0, HR)
        def _(r):
            for v in range(8):
                hist[r, pl.ds(v * L, L)] = zeros16

        cp0.wait()
        cp1.wait()
        cp2.wait()
        cp3.wait()
        base = s * XPT
        nq = XPT // K
        tail = XPT - nq * K

        def spl_src(q, n):
            return x_hbm.at[pl.ds(base + q * K, n), pl.ds(c * DH, DH)]

        pltpu.async_copy(spl_src(0, K), rows[0], gsem[0])
        pltpu.async_copy(spl_src(1, K), rows[1], gsem[1])
        for q in range(nq):
            b = q % 2
            pltpu.make_async_copy(spl_src(0, K), rows[b], gsem[b]).wait()
            pltpu.sync_copy(rows[b], xh.at[pl.ds(base + q * K, K)])
            if q + 2 < nq:
                pltpu.async_copy(spl_src(q + 2, K), rows[b], gsem[b])
        if tail:
            pltpu.sync_copy(spl_src(nq, tail), rows[0].at[pl.ds(0, tail)])
            pltpu.sync_copy(rows[0].at[pl.ds(0, tail)],
                            xh.at[pl.ds(base + nq * K, tail)])

        @pl.when(s == 0)
        def _():
            pltpu.sync_copy(hist, sdeg)
        plsc.subcore_barrier()

        def hist_row(r):
            for v in range(NFULL):
                idx = didx[pl.ds(r * K + v * L, L)]
                plsc.addupdate_scatter(hist, [idx >> 7, idx & 127], ones16)

        for b in range(NBUF):
            pltpu.async_copy(xh.at[sidx.at[pl.ds(b * K, K)]], rows[b], gsem[b])

        @pl.loop(0, NCHUNK, step=NBUF)
        def _(j):
            for b in range(NBUF):
                pltpu.make_async_copy(xh.at[sidx.at[pl.ds(0, K)]], rows[b],
                                      gsem[b]).wait()
                pltpu.async_copy(rows[b], agg.at[didx.at[pl.ds((j + b) * K, K)]],
                                 ssem[b], add=True)
            for b in range(NBUF):
                hist_row(j + b)
            for b in range(NBUF):
                pltpu.make_async_copy(rows[b], agg.at[didx.at[pl.ds(0, K)]],
                                      ssem[b]).wait()

                @pl.when(j + NBUF + b < NCHUNK)
                def _():
                    pltpu.async_copy(
                        xh.at[sidx.at[pl.ds((j + NBUF + b) * K, K)]], rows[b],
                        gsem[b])

        plsc.subcore_barrier()
        pltpu.sync_copy(hist, sdeg.at[iota_v], add=True)
        plsc.subcore_barrier()
        pltpu.sync_copy(agg.at[pl.ds(s * RPT, RPT)],
                        outa_hbm.at[pl.ds(s * RPT, RPT), pl.ds(c * DH, DH)])
        pltpu.sync_copy(sdeg.at[pl.ds(s * (HR // NS), HR // NS)],
                        outd_hbm.at[c, pl.ds(s * (HR // NS), HR // NS)])

    return body(x, edges, zeros_tile, iota_hr)


def _tc_root(x, wrT, bias):
    BR = 1024
    nblk = (N + BR - 1) // BR

    def body(x_ref, wr_ref, b_ref, o_ref):
        o_ref[...] = jnp.dot(
            x_ref[...], wr_ref[...],
            preferred_element_type=jnp.float32) + b_ref[...]

    return pl.pallas_call(
        body,
        grid=(nblk,),
        in_specs=[
            pl.BlockSpec((BR, DIN), lambda i: (i, 0)),
            pl.BlockSpec((DIN, DIN), lambda i: (0, 0)),
            pl.BlockSpec((1, DIN), lambda i: (0, 0)),
        ],
        out_specs=pl.BlockSpec((BR, DIN), lambda i: (i, 0)),
        out_shape=jax.ShapeDtypeStruct((N, DIN), jnp.float32),
    )(x, wrT, bias)


def _tc_combine(agg, deg, root, wnT):
    BR = 2048
    nblk = (N + BR - 1) // BR
    DR = BR // 128

    def body(a_ref, d_ref, r_ref, wn_ref, o_ref):
        dcol = jnp.concatenate(
            [jnp.transpose(d_ref[0, i:i + 1, :]) for i in range(DR)], axis=0)
        d = jnp.maximum(dcol, 1.0)
        neigh = a_ref[...] / d
        o_ref[...] = jnp.dot(
            neigh.astype(jnp.bfloat16), wn_ref[...],
            preferred_element_type=jnp.float32) + r_ref[...]

    return pl.pallas_call(
        body,
        grid=(nblk,),
        in_specs=[
            pl.BlockSpec((BR, DIN), lambda i: (i, 0)),
            pl.BlockSpec((1, DR, 128), lambda i: (0, i, 0)),
            pl.BlockSpec((BR, DIN), lambda i: (i, 0)),
            pl.BlockSpec((DIN, DIN), lambda i: (0, 0)),
        ],
        out_specs=pl.BlockSpec((BR, DIN), lambda i: (i, 0)),
        out_shape=jax.ShapeDtypeStruct((N, DIN), jnp.float32),
    )(agg, deg, root, wnT.astype(jnp.bfloat16))


def kernel(x, edge_index, W_neigh, b_neigh, W_root, b_root):
    f32 = jnp.float32
    edges = edge_index.astype(jnp.int32)
    x = x.astype(f32)
    zeros_tile = jnp.zeros((RPT, DH), f32)
    iota_hr = jnp.arange(HR, dtype=jnp.int32)
    agg, deg, _ = _sc_aggregate(x, edges, zeros_tile, iota_hr)
    bias = (b_neigh + b_root).reshape(1, DIN).astype(f32)
    root = _tc_root(x, W_root.T.astype(f32), bias)
    return _tc_combine(agg, deg, root, W_neigh.T.astype(f32))


# speedup vs baseline: 1.1137x; 1.0235x over previous
import dataclasses
import functools

import jax
import jax.numpy as jnp
from jax import lax
from jax.experimental import pallas as pl
from jax.experimental.pallas import tpu as pltpu
from jax.experimental.pallas import tpu_sc as plsc

N = 10000
DIN = 128
E = 320000
NC = 2
NS = 16
L = 16
DH = DIN // NC
K = 128
NBUF = 4
R = 10240
RPT = R // NS
XPT = N // NS
EPT = E // NS
NCHUNK = EPT // K
KT = EPT - NCHUNK * K
HR = 80
HPT = HR // NS

_CP = pltpu.CompilerParams(use_tc_tiling_on_sc=False)
if "needs_layout_passes" in pltpu.CompilerParams.__dataclass_fields__:
    _CP = dataclasses.replace(_CP, needs_layout_passes=False)


def _sc_aggregate(x, edges, zeros_tile):
    mesh = plsc.VectorSubcoreMesh(core_axis_name="c", subcore_axis_name="s")

    @functools.partial(
        pl.kernel,
        mesh=mesh,
        compiler_params=_CP,
        out_type=[jax.ShapeDtypeStruct((R, DIN), jnp.float32),
                  jax.ShapeDtypeStruct((NC, HR, 128), jnp.float32),
                  jax.ShapeDtypeStruct((NC, N, DH), jnp.float32)],
        scratch_types=[
            pltpu.VMEM((EPT,), jnp.int32),
            pltpu.VMEM((EPT,), jnp.int32),
            *[pltpu.VMEM((K, DH), jnp.float32) for _ in range(NBUF)],
            pltpu.VMEM((HR, 128), jnp.float32),
            pltpu.VMEM((HR,), jnp.int32),
            pltpu.VMEM_SHARED((R, DH), jnp.float32),
            pltpu.VMEM_SHARED((HR, 128), jnp.float32),
            *[pltpu.SemaphoreType.DMA for _ in range(2 * NBUF)],
        ],
    )
    def body(x_hbm, e_hbm, z_hbm, outa_hbm, outd_hbm, xh_hbm,
             sidx, didx, *bufs):
        rows = bufs[:NBUF]
        hist = bufs[NBUF]
        iota_v = bufs[NBUF + 1]
        agg = bufs[NBUF + 2]
        sdeg = bufs[NBUF + 3]
        gsem = bufs[NBUF + 4:NBUF + 4 + NBUF]
        ssem = bufs[NBUF + 4 + NBUF:]
        c = lax.axis_index("c")
        s = lax.axis_index("s")
        xh = xh_hbm.at[c]
        ones16 = jnp.ones((L,), jnp.float32)
        zeros16 = jnp.zeros((L,), jnp.float32)
        iota16 = lax.broadcasted_iota(jnp.int32, (L,), 0)

        base = s * XPT
        nq = XPT // K
        tail = XPT - nq * K

        def spl_src(q, n):
            return x_hbm.at[pl.ds(base + q * K, n), pl.ds(c * DH, DH)]

        pltpu.async_copy(spl_src(0, K), rows[0], gsem[0])
        pltpu.async_copy(spl_src(1, K), rows[1], gsem[1])
        cp0 = pltpu.async_copy(e_hbm.at[0, pl.ds(s * EPT, EPT)], sidx, ssem[0])
        cp1 = pltpu.async_copy(e_hbm.at[1, pl.ds(s * EPT, EPT)], didx, ssem[1])
        cp2 = pltpu.async_copy(z_hbm, agg.at[pl.ds(s * RPT, RPT)], ssem[2])

        @pl.loop(0, HR)
        def _(r):
            for v in range(8):
                hist[r, pl.ds(v * L, L)] = zeros16

        for q in range(HR // L):
            iota_v[pl.ds(q * L, L)] = iota16 + (q * L)

        for q in range(nq):
            b = q % 2
            pltpu.make_async_copy(spl_src(0, K), rows[b], gsem[b]).wait()
            pltpu.sync_copy(rows[b], xh.at[pl.ds(base + q * K, K)])
            if q + 2 < nq:
                pltpu.async_copy(spl_src(q + 2, K), rows[b], gsem[b])
        if tail:
            pltpu.sync_copy(spl_src(nq, tail), rows[0].at[pl.ds(0, tail)])
            pltpu.sync_copy(rows[0].at[pl.ds(0, tail)],
                            xh.at[pl.ds(base + nq * K, tail)])
        cp0.wait()
        cp1.wait()
        cp2.wait()

        @pl.when(s == 0)
        def _():
            pltpu.sync_copy(hist, sdeg)
        plsc.subcore_barrier()

        def hist_span(off, nvec):
            for v in range(nvec):
                idx = didx[pl.ds(off + v * L, L)]
                plsc.addupdate_scatter(hist, [idx >> 7, idx & 127], ones16)

        for b in range(NBUF):
            pltpu.async_copy(xh.at[sidx.at[pl.ds(b * K, K)]], rows[b], gsem[b])

        @pl.loop(0, NCHUNK, step=NBUF)
        def _(j):
            for b in range(NBUF):
                pltpu.make_async_copy(xh.at[sidx.at[pl.ds(0, K)]], rows[b],
                                      gsem[b]).wait()
                pltpu.async_copy(rows[b], agg.at[didx.at[pl.ds((j + b) * K, K)]],
                                 ssem[b], add=True)
            for b in range(NBUF):
                hist_span((j + b) * K, K // L)
            for b in range(NBUF):
                pltpu.make_async_copy(rows[b], agg.at[didx.at[pl.ds(0, K)]],
                                      ssem[b]).wait()

                @pl.when(j + NBUF + b < NCHUNK)
                def _():
                    pltpu.async_copy(
                        xh.at[sidx.at[pl.ds((j + NBUF + b) * K, K)]], rows[b],
                        gsem[b])

        if KT:
            toff = NCHUNK * K
            pltpu.async_copy(xh.at[sidx.at[pl.ds(toff, KT)]],
                             rows[0].at[pl.ds(0, KT)], gsem[0]).wait()
            pltpu.sync_copy(rows[0].at[pl.ds(0, KT)],
                            agg.at[didx.at[pl.ds(toff, KT)]], add=True)
            hist_span(toff, KT // L)

        plsc.subcore_barrier()
        pltpu.sync_copy(hist, sdeg.at[iota_v], add=True)
        plsc.subcore_barrier()
        pltpu.sync_copy(agg.at[pl.ds(s * RPT, RPT)],
                        outa_hbm.at[pl.ds(s * RPT, RPT), pl.ds(c * DH, DH)])
        pltpu.sync_copy(sdeg.at[pl.ds(s * HPT, HPT)],
                        outd_hbm.at[c, pl.ds(s * HPT, HPT)])

    return body(x, edges, zeros_tile)


def _tc_root(x, wrT, bias):
    BR = 1024
    nblk = (N + BR - 1) // BR

    def body(x_ref, wr_ref, b_ref, o_ref):
        o_ref[...] = jnp.dot(
            x_ref[...], wr_ref[...],
            preferred_element_type=jnp.float32) + b_ref[...]

    return pl.pallas_call(
        body,
        grid=(nblk,),
        in_specs=[
            pl.BlockSpec((BR, DIN), lambda i: (i, 0)),
            pl.BlockSpec((DIN, DIN), lambda i: (0, 0)),
            pl.BlockSpec((1, DIN), lambda i: (0, 0)),
        ],
        out_specs=pl.BlockSpec((BR, DIN), lambda i: (i, 0)),
        out_shape=jax.ShapeDtypeStruct((N, DIN), jnp.float32),
    )(x, wrT, bias)


def _tc_combine(agg, deg, root, wnT):
    BR = 2048
    nblk = (N + BR - 1) // BR
    DR = BR // 128

    def body(a_ref, d_ref, r_ref, wn_ref, o_ref):
        dcol = jnp.concatenate(
            [jnp.transpose(d_ref[0, i:i + 1, :]) for i in range(DR)], axis=0)
        d = jnp.maximum(dcol, 1.0)
        neigh = a_ref[...] / d
        o_ref[...] = jnp.dot(
            neigh.astype(jnp.bfloat16), wn_ref[...],
            preferred_element_type=jnp.float32) + r_ref[...]

    return pl.pallas_call(
        body,
        grid=(nblk,),
        in_specs=[
            pl.BlockSpec((BR, DIN), lambda i: (i, 0)),
            pl.BlockSpec((1, DR, 128), lambda i: (0, i, 0)),
            pl.BlockSpec((BR, DIN), lambda i: (i, 0)),
            pl.BlockSpec((DIN, DIN), lambda i: (0, 0)),
        ],
        out_specs=pl.BlockSpec((BR, DIN), lambda i: (i, 0)),
        out_shape=jax.ShapeDtypeStruct((N, DIN), jnp.float32),
    )(agg, deg, root, wnT.astype(jnp.bfloat16))


def kernel(x, edge_index, W_neigh, b_neigh, W_root, b_root):
    f32 = jnp.float32
    edges = edge_index.astype(jnp.int32)
    x = x.astype(f32)
    zeros_tile = jnp.zeros((RPT, DH), f32)
    agg, deg, _ = _sc_aggregate(x, edges, zeros_tile)
    bias = (b_neigh + b_root).reshape(1, DIN).astype(f32)
    root = _tc_root(x, W_root.T.astype(f32), bias)
    return _tc_combine(agg, deg, root, W_neigh.T.astype(f32))
